# baseline (speedup 1.0000x reference)
import functools

import jax
import jax.numpy as jnp
from jax import lax
from jax.experimental import pallas as pl
from jax.experimental.pallas import tpu as pltpu
from jax.experimental.pallas import tpu_sc as plsc

B, C, H, W = 4, 192, 24, 24
E, HID, CO = 8, 768, 192
N = B * H * W
BLK = 256
G = N // BLK + (E - 1)
CAP = G * BLK
CH = 128
NCH = N // CH
CP = 256


def _silu(x):
    return x * jax.nn.sigmoid(x)


def _front_kernel(x_hbm, dwt_ref, bn1g_ref, bn1b_ref, w1_ref, b1_ref,
                  w2t_ref, b2_ref, rw_ref, rb_ref,
                  yfp_ref, dst_ref, topp_ref, blke_ref, aux_ref,
                  xp_scr, dma_sem):
    cp = pltpu.make_async_copy(x_hbm, xp_scr.at[:, 1:1 + H, 8:8 + W, :],
                               dma_sem)
    cp.start()
    xp_scr[:, 0:1, :, :] = jnp.zeros((B, 1, W + 16, C), jnp.float32)
    xp_scr[:, H + 1:H + 2, :, :] = jnp.zeros((B, 1, W + 16, C), jnp.float32)
    xp_scr[:, :, 0:8, :] = jnp.zeros((B, H + 2, 8, C), jnp.float32)
    xp_scr[:, :, W + 8:W + 16, :] = jnp.zeros((B, H + 2, 8, C), jnp.float32)
    cp.wait()
    xp = xp_scr[...]
    y = jnp.zeros((B, H, W, C), jnp.float32)
    for dw in range(3):
        xw = xp[:, :, 7 + dw:7 + dw + W, :]
        for dh in range(3):
            tap = dwt_ref[dh, dw]
            y = y + xw[:, dh:dh + H, :, :] * tap
    m = jnp.mean(y, axis=(0, 1, 2), keepdims=True)
    v = jnp.mean((y - m) * (y - m), axis=(0, 1, 2), keepdims=True)
    g1 = bn1g_ref[...].reshape(1, 1, 1, C)
    b1 = bn1b_ref[...].reshape(1, 1, 1, C)
    y = (y - m) * jax.lax.rsqrt(v + 1e-3) * g1 + b1
    y = _silu(y)
    dnt = (((1,), (1,)), ((), ()))
    s = jnp.mean(y, axis=(1, 2))
    s = _silu(lax.dot_general(s, w1_ref[...], dnt,
                              preferred_element_type=jnp.float32)
              + b1_ref[...])
    s = jax.nn.sigmoid(jnp.dot(s, w2t_ref[...],
                               preferred_element_type=jnp.float32)
                       + b2_ref[...])
    yf = (y * s[:, None, None, :]).reshape(N, C)
    yfp_ref[:, :C] = yf
    logits = lax.dot_general(yf, rw_ref[...], dnt,
                             preferred_element_type=jnp.float32) + rb_ref[...]
    lmax = jnp.max(logits, axis=1, keepdims=True)
    ex = jnp.exp(logits - lmax)
    probs = ex / jnp.sum(ex, axis=1, keepdims=True)
    top_p = jnp.max(probs, axis=1, keepdims=True)
    lane = lax.broadcasted_iota(jnp.int32, (N, E), 1)
    top_i = jnp.min(jnp.where(probs == top_p, lane, E), axis=1,
                    keepdims=True)
    mask = (lane == top_i).astype(jnp.float32)
    counts = jnp.sum(mask, axis=0, keepdims=True)
    mean_probs = jnp.mean(probs, axis=0, keepdims=True)
    aux_ref[...] = (E * jnp.sum(mean_probs * counts * (1.0 / N))
                    ).reshape(1, 1)
    padded = jnp.floor((counts + (BLK - 1)) * (1.0 / BLK)) * BLK
    er = lax.broadcasted_iota(jnp.int32, (E, E), 0)
    ec = lax.broadcasted_iota(jnp.int32, (E, E), 1)
    tstrict = (er < ec).astype(jnp.float32)
    off = jnp.dot(padded, tstrict,
                  preferred_element_type=jnp.float32)
    off_end = off + padded
    gs = lax.broadcasted_iota(jnp.int32, (G, E), 0).astype(jnp.float32) * float(BLK)
    nfull = jnp.sum((off_end <= gs).astype(jnp.float32), axis=1,
                    keepdims=True)
    blke_ref[...] = jnp.minimum(nfull, float(E - 1)).astype(jnp.int32
                                                            ).reshape(G)
    rr = lax.broadcasted_iota(jnp.int32, (CH, CH), 0)
    rc = lax.broadcasted_iota(jnp.int32, (CH, CH), 1)
    ltri = (rr >= rc).astype(jnp.float32)
    tot = jnp.zeros((1, E), jnp.float32)
    cols = []
    for c in range(NCH):
        mblk = mask[c * CH:(c + 1) * CH, :]
        pos = jnp.dot(ltri, mblk,
                      preferred_element_type=jnp.float32) + tot
        cols.append(jnp.sum(mblk * (off + pos - 1.0), axis=1,
                            keepdims=True))
        tot = tot + jnp.sum(mblk, axis=0, keepdims=True)
    dmat = jnp.concatenate(cols, axis=1).T
    dsti = dmat.astype(jnp.int32)
    for c in range(NCH):
        dst_ref[pl.ds(c * CH, CH)] = dsti[c]
    topp_ref[...] = top_p


def _expert_kernel(blke_ref, xs_ref, w1_ref, g_ref, b_ref, w2t_ref, o_ref):
    xb = xs_ref[:, :C]
    h = jnp.dot(xb, w1_ref[0], preferred_element_type=jnp.float32)
    mu = jnp.mean(h, axis=1, keepdims=True)
    var = jnp.mean((h - mu) * (h - mu), axis=1, keepdims=True)
    h = (h - mu) * jax.lax.rsqrt(var + 1e-5) * g_ref[0] + b_ref[0]
    h = _silu(h)
    o_ref[:, :CO] = lax.dot_general(
        h, w2t_ref[0], (((1,), (1,)), ((), ())),
        preferred_element_type=jnp.float32)


def _bn2_kernel(et_ref, topp_ref, xt_ref, g_ref, b_ref, o_ref):
    sel = et_ref[:, :CO] * topp_ref[...]
    m2 = jnp.mean(sel, axis=0, keepdims=True)
    v2 = jnp.mean((sel - m2) * (sel - m2), axis=0, keepdims=True)
    o_ref[...] = ((sel - m2) * jax.lax.rsqrt(v2 + 1e-3) * g_ref[...]
                  + b_ref[...] + xt_ref[...])


def _sc_workers():
    info = plsc.get_sparse_core_info()
    return info.num_cores, info.num_subcores


def _sc_scatter(rows, idx, cap):
    n, d = rows.shape
    nc, ns = _sc_workers()
    per = n // (nc * ns)
    mesh = plsc.VectorSubcoreMesh(core_axis_name="c", subcore_axis_name="s")

    @functools.partial(
        pl.kernel, mesh=mesh,
        out_type=jax.ShapeDtypeStruct((cap, d), rows.dtype),
        scratch_types=[pltpu.VMEM((per,), jnp.int32),
                       pltpu.VMEM((per, d), rows.dtype),
                       pltpu.SemaphoreType.DMA,
                       pltpu.SemaphoreType.DMA],
    )
    def k(rows_hbm, idx_hbm, out_hbm, idx_v, rows_v, sem1, sem2):
        wid = lax.axis_index("s") * nc + lax.axis_index("c")
        base = wid * per
        cp1 = pltpu.make_async_copy(idx_hbm.at[pl.ds(base, per)], idx_v, sem1)
        cp2 = pltpu.make_async_copy(rows_hbm.at[pl.ds(base, per)], rows_v,
                                    sem2)
        cp1.start()
        cp2.start()
        cp1.wait()
        cp2.wait()
        pltpu.async_copy(rows_v, out_hbm.at[idx_v], sem1).wait()

    return k(rows, idx)


def _sc_gather(table, idx):
    n = idx.shape[0]
    d = table.shape[1]
    nc, ns = _sc_workers()
    per = n // (nc * ns)
    mesh = plsc.VectorSubcoreMesh(core_axis_name="c", subcore_axis_name="s")

    @functools.partial(
        pl.kernel, mesh=mesh,
        out_type=jax.ShapeDtypeStruct((n, d), table.dtype),
        scratch_types=[pltpu.VMEM((per,), jnp.int32),
                       pltpu.VMEM((per, d), table.dtype),
                       pltpu.SemaphoreType.DMA],
    )
    def k(table_hbm, idx_hbm, out_hbm, idx_v, rows_v, sem):
        wid = lax.axis_index("s") * nc + lax.axis_index("c")
        base = wid * per
        pltpu.sync_copy(idx_hbm.at[pl.ds(base, per)], idx_v)
        pltpu.async_copy(table_hbm.at[idx_v], rows_v, sem).wait()
        pltpu.sync_copy(rows_v, out_hbm.at[pl.ds(base, per)])

    return k(table, idx)


def kernel(x, dw_w, bn1_g, bn1_b, se_w1, se_b1, se_w2, se_b2, router_w,
           router_b, ew1, eln_g, eln_b, ew2, bn2_g, bn2_b):
    f32 = jnp.float32
    xt4 = jnp.transpose(x, (0, 2, 3, 1))

    yfp, dst1, top_p, blk_e1, aux = pl.pallas_call(
        _front_kernel,
        in_specs=[pl.BlockSpec(memory_space=pl.ANY)]
        + [pl.BlockSpec()] * 9,
        scratch_shapes=[pltpu.VMEM((B, H + 2, W + 16, C), f32),
                        pltpu.SemaphoreType.DMA],
        out_shape=[jax.ShapeDtypeStruct((N, CP), f32),
                   jax.ShapeDtypeStruct((N,), jnp.int32),
                   jax.ShapeDtypeStruct((N, 1), f32),
                   jax.ShapeDtypeStruct((G,), jnp.int32),
                   jax.ShapeDtypeStruct((1, 1), f32)],
    )(xt4, jnp.transpose(dw_w, (1, 2, 3, 0)).reshape(3, 3, C),
      bn1_g.reshape(1, C), bn1_b.reshape(1, C),
      se_w1, se_b1.reshape(1, -1), se_w2.T, se_b2.reshape(1, C),
      router_w, router_b.reshape(1, E))

    xs = _sc_scatter(yfp, dst1, CAP)

    es = pl.pallas_call(
        _expert_kernel,
        grid_spec=pltpu.PrefetchScalarGridSpec(
            num_scalar_prefetch=1,
            grid=(G,),
            in_specs=[
                pl.BlockSpec((BLK, CP), lambda g, be: (g, 0)),
                pl.BlockSpec((1, C, HID), lambda g, be: (be[g], 0, 0)),
                pl.BlockSpec((1, 1, HID), lambda g, be: (be[g], 0, 0)),
                pl.BlockSpec((1, 1, HID), lambda g, be: (be[g], 0, 0)),
                pl.BlockSpec((1, CO, HID), lambda g, be: (be[g], 0, 0)),
            ],
            out_specs=pl.BlockSpec((BLK, CP), lambda g, be: (g, 0)),
        ),
        out_shape=jax.ShapeDtypeStruct((CAP, CP), f32),
    )(blk_e1, xs, ew1, eln_g.reshape(E, 1, HID),
      eln_b.reshape(E, 1, HID), jnp.swapaxes(ew2, 1, 2))

    et = _sc_gather(es, dst1)

    out_tok = pl.pallas_call(
        _bn2_kernel,
        out_shape=jax.ShapeDtypeStruct((N, CO), f32),
    )(et, top_p, xt4.reshape(N, C), bn2_g.reshape(1, CO),
      bn2_b.reshape(1, CO))

    out = jnp.transpose(out_tok.reshape(B, H, W, CO), (0, 3, 1, 2))
    return (out, aux.reshape(()))


# speedup vs baseline: 1.0713x; 1.0713x over previous
import functools

import jax
import jax.numpy as jnp
from jax import lax
from jax.experimental import pallas as pl
from jax.experimental.pallas import tpu as pltpu
from jax.experimental.pallas import tpu_sc as plsc

B, C, H, W = 4, 192, 24, 24
E, HID, CO = 8, 768, 192
N = B * H * W
BLK = 256
G = N // BLK + (E - 1)
CAP = G * BLK
CH = 128
NCH = N // CH
CP = 256


def _silu(x):
    return x * jax.nn.sigmoid(x)


def _front_kernel(x_hbm, dwt_ref, bn1g_ref, bn1b_ref, w1_ref, b1_ref,
                  w2t_ref, b2_ref, rw_ref, rb_ref,
                  yfp_ref, dst_ref, topp_ref, blke_ref, aux_ref,
                  xp_scr, xw_scr, dma_sem):
    cp = pltpu.make_async_copy(x_hbm, xp_scr.at[:, 1:1 + H, 8:8 + W, :],
                               dma_sem)
    cp.start()
    xp_scr[:, 0:1, :, :] = jnp.zeros((B, 1, W + 16, C), jnp.float32)
    xp_scr[:, H + 1:H + 2, :, :] = jnp.zeros((B, 1, W + 16, C), jnp.float32)
    xp_scr[:, :, 0:8, :] = jnp.zeros((B, H + 2, 8, C), jnp.float32)
    xp_scr[:, :, W + 8:W + 16, :] = jnp.zeros((B, H + 2, 8, C), jnp.float32)
    cp.wait()
    xp = xp_scr[...]
    y = None
    for dw in range(3):
        if dw == 1:
            xwv = xp[:, :, 8:8 + W, :]
        else:
            xw_scr[...] = xp[:, :, 7 + dw:7 + dw + W, :]
            xwv = xw_scr[...]
        for dh in range(3):
            tap = dwt_ref[dh, dw]
            t = xwv[:, dh:dh + H, :, :] * tap
            y = t if y is None else y + t
    yr = y.reshape(N, C)
    onesn = jnp.full((1, N), 1.0, jnp.float32)
    m = jnp.dot(onesn, yr, preferred_element_type=jnp.float32) * (1.0 / N)
    msq = jnp.dot(onesn, yr * yr,
                  preferred_element_type=jnp.float32) * (1.0 / N)
    v = msq - m * m
    y2 = (yr - m) * jax.lax.rsqrt(v + 1e-3) * bn1g_ref[...] + bn1b_ref[...]
    y2 = _silu(y2)
    dnt = (((1,), (1,)), ((), ()))
    bn_r = lax.broadcasted_iota(jnp.int32, (B, N), 0)
    bn_c = lax.broadcasted_iota(jnp.int32, (B, N), 1)
    msel = (bn_c // (N // B) == bn_r).astype(jnp.float32)
    s = jnp.dot(msel, y2, preferred_element_type=jnp.float32) * (B / N)
    s = _silu(lax.dot_general(s, w1_ref[...], dnt,
                              preferred_element_type=jnp.float32)
              + b1_ref[...])
    s = jax.nn.sigmoid(jnp.dot(s, w2t_ref[...],
                               preferred_element_type=jnp.float32)
                       + b2_ref[...])
    srows = lax.dot_general(msel, s, (((0,), (0,)), ((), ())),
                            preferred_element_type=jnp.float32)
    yf = y2 * srows
    yfp_ref[:, :C] = yf
    logits = lax.dot_general(yf, rw_ref[...], dnt,
                             preferred_element_type=jnp.float32) + rb_ref[...]
    lmax = jnp.max(logits, axis=1, keepdims=True)
    ex = jnp.exp(logits - lmax)
    probs = ex / jnp.sum(ex, axis=1, keepdims=True)
    top_p = jnp.max(probs, axis=1, keepdims=True)
    lane = lax.broadcasted_iota(jnp.int32, (N, E), 1)
    top_i = jnp.min(jnp.where(probs == top_p, lane, E), axis=1,
                    keepdims=True)
    mask = (lane == top_i).astype(jnp.float32)
    counts = jnp.sum(mask, axis=0, keepdims=True)
    mean_probs = jnp.mean(probs, axis=0, keepdims=True)
    aux_ref[...] = (E * jnp.sum(mean_probs * counts * (1.0 / N))
                    ).reshape(1, 1)
    padded = jnp.floor((counts + (BLK - 1)) * (1.0 / BLK)) * BLK
    er = lax.broadcasted_iota(jnp.int32, (E, E), 0)
    ec = lax.broadcasted_iota(jnp.int32, (E, E), 1)
    tstrict = (er < ec).astype(jnp.float32)
    off = jnp.dot(padded, tstrict,
                  preferred_element_type=jnp.float32)
    off_end = off + padded
    gs = lax.broadcasted_iota(jnp.int32, (G, E), 0).astype(jnp.float32) * float(BLK)
    nfull = jnp.sum((off_end <= gs).astype(jnp.float32), axis=1,
                    keepdims=True)
    blke_ref[...] = jnp.minimum(nfull, float(E - 1)).astype(jnp.int32
                                                            ).reshape(G)
    rr = lax.broadcasted_iota(jnp.int32, (CH, CH), 0)
    rc = lax.broadcasted_iota(jnp.int32, (CH, CH), 1)
    ltri = (rr >= rc).astype(jnp.float32)
    tot = jnp.zeros((1, E), jnp.float32)
    cols = []
    for c in range(NCH):
        mblk = mask[c * CH:(c + 1) * CH, :]
        pos = jnp.dot(ltri, mblk,
                      preferred_element_type=jnp.float32) + tot
        cols.append(jnp.sum(mblk * (off + pos - 1.0), axis=1,
                            keepdims=True))
        tot = tot + jnp.sum(mblk, axis=0, keepdims=True)
    dmat = jnp.concatenate(cols, axis=1).T
    dsti = dmat.astype(jnp.int32)
    for c in range(NCH):
        dst_ref[pl.ds(c * CH, CH)] = dsti[c]
    topp_ref[...] = top_p


def _expert_kernel(blke_ref, xs_ref, w1_ref, g_ref, b_ref, w2t_ref, o_ref):
    xb = xs_ref[:, :C]
    h = jnp.dot(xb, w1_ref[0], preferred_element_type=jnp.float32)
    mu = jnp.mean(h, axis=1, keepdims=True)
    var = jnp.mean((h - mu) * (h - mu), axis=1, keepdims=True)
    h = (h - mu) * jax.lax.rsqrt(var + 1e-5) * g_ref[0] + b_ref[0]
    h = _silu(h)
    o_ref[:, :CO] = lax.dot_general(
        h, w2t_ref[0], (((1,), (1,)), ((), ())),
        preferred_element_type=jnp.float32)


def _bn2_kernel(et_ref, topp_ref, xt_ref, g_ref, b_ref, o_ref):
    sel = et_ref[:, :CO] * topp_ref[...]
    m2 = jnp.mean(sel, axis=0, keepdims=True)
    v2 = jnp.mean((sel - m2) * (sel - m2), axis=0, keepdims=True)
    o_ref[...] = ((sel - m2) * jax.lax.rsqrt(v2 + 1e-3) * g_ref[...]
                  + b_ref[...] + xt_ref[...])


def _sc_workers():
    info = plsc.get_sparse_core_info()
    return info.num_cores, info.num_subcores


def _sc_scatter(rows, idx, cap):
    n, d = rows.shape
    nc, ns = _sc_workers()
    per = n // (nc * ns)
    mesh = plsc.VectorSubcoreMesh(core_axis_name="c", subcore_axis_name="s")

    @functools.partial(
        pl.kernel, mesh=mesh,
        out_type=jax.ShapeDtypeStruct((cap, d), rows.dtype),
        scratch_types=[pltpu.VMEM((per,), jnp.int32),
                       pltpu.VMEM((per, d), rows.dtype),
                       pltpu.SemaphoreType.DMA,
                       pltpu.SemaphoreType.DMA],
    )
    def k(rows_hbm, idx_hbm, out_hbm, idx_v, rows_v, sem1, sem2):
        wid = lax.axis_index("s") * nc + lax.axis_index("c")
        base = wid * per
        cp1 = pltpu.make_async_copy(idx_hbm.at[pl.ds(base, per)], idx_v, sem1)
        cp2 = pltpu.make_async_copy(rows_hbm.at[pl.ds(base, per)], rows_v,
                                    sem2)
        cp1.start()
        cp2.start()
        cp1.wait()
        cp2.wait()
        pltpu.async_copy(rows_v, out_hbm.at[idx_v], sem1).wait()

    return k(rows, idx)


def _sc_gather(table, idx):
    n = idx.shape[0]
    d = table.shape[1]
    nc, ns = _sc_workers()
    per = n // (nc * ns)
    mesh = plsc.VectorSubcoreMesh(core_axis_name="c", subcore_axis_name="s")

    @functools.partial(
        pl.kernel, mesh=mesh,
        out_type=jax.ShapeDtypeStruct((n, d), table.dtype),
        scratch_types=[pltpu.VMEM((per,), jnp.int32),
                       pltpu.VMEM((per, d), table.dtype),
                       pltpu.SemaphoreType.DMA],
    )
    def k(table_hbm, idx_hbm, out_hbm, idx_v, rows_v, sem):
        wid = lax.axis_index("s") * nc + lax.axis_index("c")
        base = wid * per
        pltpu.sync_copy(idx_hbm.at[pl.ds(base, per)], idx_v)
        pltpu.async_copy(table_hbm.at[idx_v], rows_v, sem).wait()
        pltpu.sync_copy(rows_v, out_hbm.at[pl.ds(base, per)])

    return k(table, idx)


def kernel(x, dw_w, bn1_g, bn1_b, se_w1, se_b1, se_w2, se_b2, router_w,
           router_b, ew1, eln_g, eln_b, ew2, bn2_g, bn2_b):
    f32 = jnp.float32
    xt4 = jnp.transpose(x, (0, 2, 3, 1))

    yfp, dst1, top_p, blk_e1, aux = pl.pallas_call(
        _front_kernel,
        in_specs=[pl.BlockSpec(memory_space=pl.ANY)]
        + [pl.BlockSpec()] * 9,
        scratch_shapes=[pltpu.VMEM((B, H + 2, W + 16, C), f32),
                        pltpu.VMEM((B, H + 2, W, C), f32),
                        pltpu.SemaphoreType.DMA],
        out_shape=[jax.ShapeDtypeStruct((N, CP), f32),
                   jax.ShapeDtypeStruct((N,), jnp.int32),
                   jax.ShapeDtypeStruct((N, 1), f32),
                   jax.ShapeDtypeStruct((G,), jnp.int32),
                   jax.ShapeDtypeStruct((1, 1), f32)],
    )(xt4, jnp.transpose(dw_w, (1, 2, 3, 0)).reshape(3, 3, C),
      bn1_g.reshape(1, C), bn1_b.reshape(1, C),
      se_w1, se_b1.reshape(1, -1), se_w2.T, se_b2.reshape(1, C),
      router_w, router_b.reshape(1, E))

    xs = _sc_scatter(yfp, dst1, CAP)

    es = pl.pallas_call(
        _expert_kernel,
        grid_spec=pltpu.PrefetchScalarGridSpec(
            num_scalar_prefetch=1,
            grid=(G,),
            in_specs=[
                pl.BlockSpec((BLK, CP), lambda g, be: (g, 0)),
                pl.BlockSpec((1, C, HID), lambda g, be: (be[g], 0, 0)),
                pl.BlockSpec((1, 1, HID), lambda g, be: (be[g], 0, 0)),
                pl.BlockSpec((1, 1, HID), lambda g, be: (be[g], 0, 0)),
                pl.BlockSpec((1, CO, HID), lambda g, be: (be[g], 0, 0)),
            ],
            out_specs=pl.BlockSpec((BLK, CP), lambda g, be: (g, 0)),
        ),
        out_shape=jax.ShapeDtypeStruct((CAP, CP), f32),
    )(blk_e1, xs, ew1, eln_g.reshape(E, 1, HID),
      eln_b.reshape(E, 1, HID), jnp.swapaxes(ew2, 1, 2))

    et = _sc_gather(es, dst1)

    out_tok = pl.pallas_call(
        _bn2_kernel,
        out_shape=jax.ShapeDtypeStruct((N, CO), f32),
    )(et, top_p, xt4.reshape(N, C), bn2_g.reshape(1, CO),
      bn2_b.reshape(1, CO))

    out = jnp.transpose(out_tok.reshape(B, H, W, CO), (0, 3, 1, 2))
    return (out, aux.reshape(()))
